# baseline (speedup 1.0000x reference)
import functools

import jax
import jax.numpy as jnp
from jax import lax
from jax.experimental import pallas as pl
from jax.experimental.pallas import tpu as pltpu
from jax.experimental.pallas import tpu_sc as plsc

D = 64
B = 16384
V = 1000000

NC = 2
NS = 16
NW = NC * NS
B_PER_W = B // NW
CHUNK = 128
NCHUNK = B_PER_W // CHUNK

H = 524288
TCOL = 1024
NRB = H // TCOL
LASTB = V // TCOL - 1


def _tc_transpose(tableT):
    def tr_kernel(xl_ref, xr_ref, o_ref):
        o_ref[:, :D] = xl_ref[...].T
        o_ref[:, D:] = xr_ref[...].T

    return pl.pallas_call(
        tr_kernel,
        grid=(NRB,),
        in_specs=[
            pl.BlockSpec((D, TCOL), lambda i: (0, i)),
            pl.BlockSpec((D, TCOL), lambda i: (0, jnp.minimum(i + NRB, LASTB))),
        ],
        out_specs=pl.BlockSpec((TCOL, 2 * D), lambda i: (i, 0)),
        out_shape=jax.ShapeDtypeStruct((H, 2 * D), jnp.float32),
    )(tableT, tableT)


def _sc_gather(ids_3d, table2):
    mesh = plsc.VectorSubcoreMesh(core_axis_name="c", subcore_axis_name="s")

    @functools.partial(
        pl.kernel,
        out_type=jax.ShapeDtypeStruct((B, 2 * D), jnp.float32),
        mesh=mesh,
        scratch_types=[
            pltpu.VMEM((NCHUNK, CHUNK), jnp.int32),
            pltpu.VMEM((B_PER_W, 2 * D), jnp.float32),
            pltpu.SemaphoreType.DMA,
        ],
    )
    def gather_kernel(ids_hbm, table_hbm, out_hbm, idx_v, rows_v, sem):
        wid = lax.axis_index("s") * NC + lax.axis_index("c")
        base = wid * B_PER_W
        pltpu.sync_copy(ids_hbm.at[wid], idx_v)
        copies = []
        for j in range(NCHUNK):
            copies.append(
                pltpu.async_copy(
                    table_hbm.at[idx_v.at[j]],
                    rows_v.at[pl.ds(j * CHUNK, CHUNK)],
                    sem,
                )
            )
        for c in copies:
            c.wait()
        pltpu.sync_copy(rows_v, out_hbm.at[pl.ds(base, B_PER_W)])

    return gather_kernel(ids_3d, table2)


BM = 2048


def _tc_select_linear(x2, par, wt, b2d):
    def mm_kernel(x_ref, p_ref, wt_ref, b_ref, o_ref):
        x2v = x_ref[...]
        x = jnp.where(p_ref[...] == 1, x2v[:, D:], x2v[:, :D])
        acc = jnp.dot(x, wt_ref[...], preferred_element_type=jnp.float32)
        o_ref[...] = jnp.maximum(acc + b_ref[...], 0.0)

    return pl.pallas_call(
        mm_kernel,
        grid=(B // BM,),
        in_specs=[
            pl.BlockSpec((BM, 2 * D), lambda i: (i, 0)),
            pl.BlockSpec((BM, 1), lambda i: (i, 0)),
            pl.BlockSpec((D, D), lambda i: (0, 0)),
            pl.BlockSpec((1, D), lambda i: (0, 0)),
        ],
        out_specs=pl.BlockSpec((BM, D), lambda i: (i, 0)),
        out_shape=jax.ShapeDtypeStruct((B, D), jnp.float32),
    )(x2, par, wt, b2d)


def kernel(ids, table, W, b):
    ids32 = ids.astype(jnp.int32)
    pair_idx = (ids32 & (H - 1)).reshape(NW, NCHUNK, CHUNK)
    half = (ids32 >= H).astype(jnp.int32).reshape(B, 1)
    table2 = _tc_transpose(table.T)
    gathered = _sc_gather(pair_idx, table2)
    return _tc_select_linear(gathered, half, W.T, b.reshape(1, D))


# speedup vs baseline: 1.6388x; 1.6388x over previous
import functools

import jax
import jax.numpy as jnp
from jax import lax
from jax.experimental import pallas as pl
from jax.experimental.pallas import tpu as pltpu
from jax.experimental.pallas import tpu_sc as plsc

D = 64
B = 16384
V = 1000000

NC = 2
NS = 16
NW = NC * NS
B_PER_W = B // NW
CHUNK = 128
NCHUNK = B_PER_W // CHUNK

H = 524288
TCOL = 4096
NRB = H // TCOL
LASTB = (V + TCOL - 1) // TCOL - 1


def _tc_transpose(tableT):
    def tr_kernel(xl_ref, xr_ref, o_ref):
        o_ref[:, :D] = xl_ref[...].T
        o_ref[:, D:] = xr_ref[...].T

    return pl.pallas_call(
        tr_kernel,
        grid=(NRB,),
        in_specs=[
            pl.BlockSpec((D, TCOL), lambda i: (0, i)),
            pl.BlockSpec((D, TCOL), lambda i: (0, jnp.minimum(i + NRB, LASTB))),
        ],
        out_specs=pl.BlockSpec((TCOL, 2 * D), lambda i: (i, 0)),
        out_shape=jax.ShapeDtypeStruct((H, 2 * D), jnp.float32),
    )(tableT, tableT)


def _sc_gather(ids_3d, table2):
    mesh = plsc.VectorSubcoreMesh(core_axis_name="c", subcore_axis_name="s")

    @functools.partial(
        pl.kernel,
        out_type=jax.ShapeDtypeStruct((B, 2 * D), jnp.float32),
        mesh=mesh,
        scratch_types=[
            pltpu.VMEM((NCHUNK, CHUNK), jnp.int32),
            pltpu.VMEM((B_PER_W, 2 * D), jnp.float32),
            pltpu.SemaphoreType.DMA,
        ],
    )
    def gather_kernel(ids_hbm, table_hbm, out_hbm, idx_v, rows_v, sem):
        wid = lax.axis_index("s") * NC + lax.axis_index("c")
        base = wid * B_PER_W
        pltpu.sync_copy(ids_hbm.at[wid], idx_v)
        copies = []
        for j in range(NCHUNK):
            copies.append(
                pltpu.async_copy(
                    table_hbm.at[idx_v.at[j]],
                    rows_v.at[pl.ds(j * CHUNK, CHUNK)],
                    sem,
                )
            )
        for c in copies:
            c.wait()
        pltpu.sync_copy(rows_v, out_hbm.at[pl.ds(base, B_PER_W)])

    return gather_kernel(ids_3d, table2)


BM = 2048


def _tc_select_linear(x2, par, wt, b2d):
    def mm_kernel(x_ref, p_ref, wt_ref, b_ref, o_ref):
        x2v = x_ref[...]
        x = jnp.where(p_ref[...] == 1, x2v[:, D:], x2v[:, :D])
        acc = jnp.dot(x, wt_ref[...], preferred_element_type=jnp.float32)
        o_ref[...] = jnp.maximum(acc + b_ref[...], 0.0)

    return pl.pallas_call(
        mm_kernel,
        grid=(B // BM,),
        in_specs=[
            pl.BlockSpec((BM, 2 * D), lambda i: (i, 0)),
            pl.BlockSpec((BM, 1), lambda i: (i, 0)),
            pl.BlockSpec((D, D), lambda i: (0, 0)),
            pl.BlockSpec((1, D), lambda i: (0, 0)),
        ],
        out_specs=pl.BlockSpec((BM, D), lambda i: (i, 0)),
        out_shape=jax.ShapeDtypeStruct((B, D), jnp.float32),
    )(x2, par, wt, b2d)


def kernel(ids, table, W, b):
    ids32 = ids.astype(jnp.int32)
    pair_idx = (ids32 & (H - 1)).reshape(NW, NCHUNK, CHUNK)
    half = (ids32 >= H).astype(jnp.int32).reshape(B, 1)
    table2 = _tc_transpose(table.T)
    gathered = _sc_gather(pair_idx, table2)
    return _tc_select_linear(gathered, half, W.T, b.reshape(1, D))


# speedup vs baseline: 1.8547x; 1.1317x over previous
import functools

import jax
import jax.numpy as jnp
from jax import lax
from jax.experimental import pallas as pl
from jax.experimental.pallas import tpu as pltpu
from jax.experimental.pallas import tpu_sc as plsc

D = 64
B = 16384
V = 1000000

NC = 2
NS = 16
NW = NC * NS
B_PER_W = B // NW
CHUNK = 128
NCHUNK = B_PER_W // CHUNK

H = 524288
TCOL = 8192
NRB = H // TCOL
LASTB = (V + TCOL - 1) // TCOL - 1


def _tc_z(tableT, W, b2d):
    def z_kernel(xl_ref, xr_ref, w_ref, b_ref, o_ref):
        dn = (((0,), (1,)), ((), ()))
        zl = lax.dot_general(
            xl_ref[...], w_ref[...], dn, preferred_element_type=jnp.float32
        )
        zr = lax.dot_general(
            xr_ref[...], w_ref[...], dn, preferred_element_type=jnp.float32
        )
        o_ref[:, :D] = zl + b_ref[...]
        o_ref[:, D:] = zr + b_ref[...]

    return pl.pallas_call(
        z_kernel,
        grid=(NRB,),
        in_specs=[
            pl.BlockSpec((D, TCOL), lambda i: (0, i)),
            pl.BlockSpec((D, TCOL), lambda i: (0, jnp.minimum(i + NRB, LASTB))),
            pl.BlockSpec((D, D), lambda i: (0, 0)),
            pl.BlockSpec((1, D), lambda i: (0, 0)),
        ],
        out_specs=pl.BlockSpec((TCOL, 2 * D), lambda i: (i, 0)),
        out_shape=jax.ShapeDtypeStruct((H, 2 * D), jnp.float32),
    )(tableT, tableT, W, b2d)


def _sc_gather(ids_3d, z2):
    mesh = plsc.VectorSubcoreMesh(core_axis_name="c", subcore_axis_name="s")

    @functools.partial(
        pl.kernel,
        out_type=jax.ShapeDtypeStruct((B, 2 * D), jnp.float32),
        mesh=mesh,
        scratch_types=[
            pltpu.VMEM((NCHUNK, CHUNK), jnp.int32),
            pltpu.VMEM((B_PER_W, 2 * D), jnp.float32),
            pltpu.SemaphoreType.DMA,
        ],
    )
    def gather_kernel(ids_hbm, table_hbm, out_hbm, idx_v, rows_v, sem):
        wid = lax.axis_index("s") * NC + lax.axis_index("c")
        base = wid * B_PER_W
        pltpu.sync_copy(ids_hbm.at[wid], idx_v)
        copies = []
        for j in range(NCHUNK):
            copies.append(
                pltpu.async_copy(
                    table_hbm.at[idx_v.at[j]],
                    rows_v.at[pl.ds(j * CHUNK, CHUNK)],
                    sem,
                )
            )
        for c in copies:
            c.wait()
        pltpu.sync_copy(rows_v, out_hbm.at[pl.ds(base, B_PER_W)])

    return gather_kernel(ids_3d, z2)


BM = 4096


def _tc_select_relu(x2, half):
    def sel_kernel(x_ref, h_ref, o_ref):
        x2v = x_ref[...]
        x = jnp.where(h_ref[...] == 1, x2v[:, D:], x2v[:, :D])
        o_ref[...] = jnp.maximum(x, 0.0)

    return pl.pallas_call(
        sel_kernel,
        grid=(B // BM,),
        in_specs=[
            pl.BlockSpec((BM, 2 * D), lambda i: (i, 0)),
            pl.BlockSpec((BM, 1), lambda i: (i, 0)),
        ],
        out_specs=pl.BlockSpec((BM, D), lambda i: (i, 0)),
        out_shape=jax.ShapeDtypeStruct((B, D), jnp.float32),
    )(x2, half)


def kernel(ids, table, W, b):
    ids32 = ids.astype(jnp.int32)
    pair_idx = (ids32 & (H - 1)).reshape(NW, NCHUNK, CHUNK)
    half = (ids32 >= H).astype(jnp.int32).reshape(B, 1)
    z2 = _tc_z(table.T, W, b.reshape(1, D))
    gathered = _sc_gather(pair_idx, z2)
    return _tc_select_relu(gathered, half)


# speedup vs baseline: 2.1087x; 1.1369x over previous
import functools

import jax
import jax.numpy as jnp
from jax import lax
from jax.experimental import pallas as pl
from jax.experimental.pallas import tpu as pltpu
from jax.experimental.pallas import tpu_sc as plsc

D = 64
B = 16384
V = 1000000

NC = 2
NS = 16
NW = NC * NS
B_PER_W = B // NW
CHUNK = 128
NCHUNK = B_PER_W // CHUNK

Q = 262144
TC2 = 4096
NRB = Q // TC2
LASTB = (V + TC2 - 1) // TC2 - 1


def _tc_z(tableT, W, b2d):
    def z_kernel(x0_ref, x1_ref, x2_ref, x3_ref, w_ref, b_ref, o_ref):
        dn = (((0,), (1,)), ((), ()))

        def zbits(x_ref):
            acc = lax.dot_general(
                x_ref[...], w_ref[...], dn, preferred_element_type=jnp.float32
            )
            zb = (acc + b_ref[...]).astype(jnp.bfloat16)
            return lax.bitcast_convert_type(zb, jnp.uint16).astype(jnp.uint32)

        w0 = zbits(x0_ref) | (zbits(x1_ref) << 16)
        w1 = zbits(x2_ref) | (zbits(x3_ref) << 16)
        o_ref[:, :D] = lax.bitcast_convert_type(w0, jnp.int32)
        o_ref[:, D:] = lax.bitcast_convert_type(w1, jnp.int32)

    return pl.pallas_call(
        z_kernel,
        grid=(NRB,),
        in_specs=[
            pl.BlockSpec((D, TC2), lambda i: (0, i)),
            pl.BlockSpec((D, TC2), lambda i: (0, i + NRB)),
            pl.BlockSpec((D, TC2), lambda i: (0, i + 2 * NRB)),
            pl.BlockSpec((D, TC2), lambda i: (0, jnp.minimum(i + 3 * NRB, LASTB))),
            pl.BlockSpec((D, D), lambda i: (0, 0)),
            pl.BlockSpec((1, D), lambda i: (0, 0)),
        ],
        out_specs=pl.BlockSpec((TC2, 2 * D), lambda i: (i, 0)),
        out_shape=jax.ShapeDtypeStruct((Q, 2 * D), jnp.int32),
    )(tableT, tableT, tableT, tableT, W, b2d)


def _sc_gather(ids_3d, zp):
    mesh = plsc.VectorSubcoreMesh(core_axis_name="c", subcore_axis_name="s")

    @functools.partial(
        pl.kernel,
        out_type=jax.ShapeDtypeStruct((B, 2 * D), jnp.int32),
        mesh=mesh,
        scratch_types=[
            pltpu.VMEM((NCHUNK, CHUNK), jnp.int32),
            pltpu.VMEM((B_PER_W, 2 * D), jnp.int32),
            pltpu.SemaphoreType.DMA,
        ],
    )
    def gather_kernel(ids_hbm, table_hbm, out_hbm, idx_v, rows_v, sem):
        wid = lax.axis_index("s") * NC + lax.axis_index("c")
        base = wid * B_PER_W
        pltpu.sync_copy(ids_hbm.at[wid], idx_v)
        copies = []
        for j in range(NCHUNK):
            copies.append(
                pltpu.async_copy(
                    table_hbm.at[idx_v.at[j]],
                    rows_v.at[pl.ds(j * CHUNK, CHUNK)],
                    sem,
                )
            )
        for c in copies:
            c.wait()
        pltpu.sync_copy(rows_v, out_hbm.at[pl.ds(base, B_PER_W)])

    return gather_kernel(ids_3d, zp)


BM = 4096


def _tc_select_relu(xp, s1, h1):
    def sel_kernel(x_ref, s_ref, h_ref, o_ref):
        xv = x_ref[...]
        xi = jnp.where(h_ref[...] == 1, xv[:, D:], xv[:, :D])
        bits = jnp.where(s_ref[...] == 1, xi >> 16, xi)
        f32 = lax.bitcast_convert_type(bits << 16, jnp.float32)
        o_ref[...] = jnp.maximum(f32, 0.0)

    return pl.pallas_call(
        sel_kernel,
        grid=(B // BM,),
        in_specs=[
            pl.BlockSpec((BM, 2 * D), lambda i: (i, 0)),
            pl.BlockSpec((BM, 1), lambda i: (i, 0)),
            pl.BlockSpec((BM, 1), lambda i: (i, 0)),
        ],
        out_specs=pl.BlockSpec((BM, D), lambda i: (i, 0)),
        out_shape=jax.ShapeDtypeStruct((B, D), jnp.float32),
    )(xp, s1, h1)


def kernel(ids, table, W, b):
    ids32 = ids.astype(jnp.int32)
    quad_idx = (ids32 & (Q - 1)).reshape(NW, NCHUNK, CHUNK)
    s1 = ((ids32 >> 18) & 1).reshape(B, 1)
    h1 = (ids32 >> 19).reshape(B, 1)
    zp = _tc_z(table.T, W, b.reshape(1, D))
    gathered = _sc_gather(quad_idx, zp)
    return _tc_select_relu(gathered, s1, h1)


# speedup vs baseline: 2.1854x; 1.0364x over previous
import functools

import jax
import jax.numpy as jnp
from jax import lax
from jax.experimental import pallas as pl
from jax.experimental.pallas import tpu as pltpu
from jax.experimental.pallas import tpu_sc as plsc

D = 64
B = 16384
V = 1000000

NC = 2
NS = 16
NW = NC * NS
B_PER_W = B // NW
CHUNK = 128
NCHUNK = B_PER_W // CHUNK

Q = 262144
TC2 = 8192
NRB = Q // TC2
LASTB = (V + TC2 - 1) // TC2 - 1


def _tc_z(tableT, W, b2d):
    def z_kernel(x0_ref, x1_ref, x2_ref, x3_ref, w_ref, b_ref, o_ref):
        dn = (((0,), (1,)), ((), ()))

        def zbits(x_ref):
            acc = lax.dot_general(
                x_ref[...], w_ref[...], dn, preferred_element_type=jnp.float32
            )
            zb = (acc + b_ref[...]).astype(jnp.bfloat16)
            return lax.bitcast_convert_type(zb, jnp.uint16).astype(jnp.uint32)

        w0 = zbits(x0_ref) | (zbits(x1_ref) << 16)
        w1 = zbits(x2_ref) | (zbits(x3_ref) << 16)
        o_ref[:, :D] = lax.bitcast_convert_type(w0, jnp.int32)
        o_ref[:, D:] = lax.bitcast_convert_type(w1, jnp.int32)

    return pl.pallas_call(
        z_kernel,
        grid=(NRB,),
        in_specs=[
            pl.BlockSpec((D, TC2), lambda i: (0, i)),
            pl.BlockSpec((D, TC2), lambda i: (0, i + NRB)),
            pl.BlockSpec((D, TC2), lambda i: (0, i + 2 * NRB)),
            pl.BlockSpec((D, TC2), lambda i: (0, jnp.minimum(i + 3 * NRB, LASTB))),
            pl.BlockSpec((D, D), lambda i: (0, 0)),
            pl.BlockSpec((1, D), lambda i: (0, 0)),
        ],
        out_specs=pl.BlockSpec((TC2, 2 * D), lambda i: (i, 0)),
        out_shape=jax.ShapeDtypeStruct((Q, 2 * D), jnp.int32),
    )(tableT, tableT, tableT, tableT, W, b2d)


def _sc_gather(ids_3d, zp):
    mesh = plsc.VectorSubcoreMesh(core_axis_name="c", subcore_axis_name="s")

    @functools.partial(
        pl.kernel,
        out_type=jax.ShapeDtypeStruct((B, 2 * D), jnp.int32),
        mesh=mesh,
        scratch_types=[
            pltpu.VMEM((NCHUNK, CHUNK), jnp.int32),
            pltpu.VMEM((B_PER_W, 2 * D), jnp.int32),
            pltpu.SemaphoreType.DMA,
        ],
    )
    def gather_kernel(ids_hbm, table_hbm, out_hbm, idx_v, rows_v, sem):
        wid = lax.axis_index("s") * NC + lax.axis_index("c")
        base = wid * B_PER_W
        pltpu.sync_copy(ids_hbm.at[wid], idx_v)
        copies = []
        for j in range(NCHUNK):
            copies.append(
                pltpu.async_copy(
                    table_hbm.at[idx_v.at[j]],
                    rows_v.at[pl.ds(j * CHUNK, CHUNK)],
                    sem,
                )
            )
        for c in copies:
            c.wait()
        pltpu.sync_copy(rows_v, out_hbm.at[pl.ds(base, B_PER_W)])

    return gather_kernel(ids_3d, zp)


BM = 4096


def _tc_select_relu(xp, s1, h1):
    def sel_kernel(x_ref, s_ref, h_ref, o_ref):
        xv = x_ref[...]
        xi = jnp.where(h_ref[...] == 1, xv[:, D:], xv[:, :D])
        bits = jnp.where(s_ref[...] == 1, xi >> 16, xi)
        f32 = lax.bitcast_convert_type(bits << 16, jnp.float32)
        o_ref[...] = jnp.maximum(f32, 0.0)

    return pl.pallas_call(
        sel_kernel,
        grid=(B // BM,),
        in_specs=[
            pl.BlockSpec((BM, 2 * D), lambda i: (i, 0)),
            pl.BlockSpec((BM, 1), lambda i: (i, 0)),
            pl.BlockSpec((BM, 1), lambda i: (i, 0)),
        ],
        out_specs=pl.BlockSpec((BM, D), lambda i: (i, 0)),
        out_shape=jax.ShapeDtypeStruct((B, D), jnp.float32),
    )(xp, s1, h1)


def kernel(ids, table, W, b):
    ids32 = ids.astype(jnp.int32)
    quad_idx = (ids32 & (Q - 1)).reshape(NW, NCHUNK, CHUNK)
    s1 = ((ids32 >> 18) & 1).reshape(B, 1)
    h1 = (ids32 >> 19).reshape(B, 1)
    zp = _tc_z(table.T, W, b.reshape(1, D))
    gathered = _sc_gather(quad_idx, zp)
    return _tc_select_relu(gathered, s1, h1)
